# Initial kernel scaffold; baseline (speedup 1.0000x reference)
import functools

import jax
import jax.numpy as jnp
from jax import lax
from jax.experimental import pallas as pl
from jax.experimental.pallas import tpu as pltpu
from jax.experimental.pallas import tpu_sc as plsc

N = 50000
E = 800000
D = 64
H = 32
NP = 50176
EP = 802816
EC = 50176
K = 512
NCHUNK = EC // K
ROWS_PER_TILE = NP // 16
ZROWS = 392


def _proj_body(x_ref, w1a_ref, w1b_ref, y_ref):
    x = x_ref[...]
    y_ref[0] = jnp.dot(x, w1a_ref[...], preferred_element_type=jnp.float32)
    y_ref[1] = jnp.dot(x, w1b_ref[...], preferred_element_type=jnp.float32)


def _project(xp, w1a, w1b):
    return pl.pallas_call(
        _proj_body,
        grid=(NP // 1024,),
        in_specs=[
            pl.BlockSpec((1024, D), lambda i: (i, 0)),
            pl.BlockSpec((D, H), lambda i: (0, 0)),
            pl.BlockSpec((D, H), lambda i: (0, 0)),
        ],
        out_specs=pl.BlockSpec((2, 1024, H), lambda i: (0, i, 0)),
        out_shape=jax.ShapeDtypeStruct((2, NP, H), jnp.float32),
    )(xp, w1a, w1b)


def _sc_body(y_hbm, src_hbm, dst_hbm, a_hbm, deg_hbm,
             acc, histsp, srcbuf, dstbuf, histv, msg, rowidx, zbuf, sem):
    c = lax.axis_index("c")
    s = lax.axis_index("s")
    z16 = jnp.zeros((16,), jnp.float32)
    iota16 = lax.iota(jnp.int32, 16)

    def zero_hist(r, _):
        for k in range(8):
            histv[r, pl.ds(k * 16, 16)] = z16
        return _
    lax.fori_loop(0, 512, zero_hist, None)

    def zero_z(r, _):
        zbuf[r, pl.ds(0, 16)] = z16
        zbuf[r, pl.ds(16, 16)] = z16
        return _
    lax.fori_loop(0, ZROWS, zero_z, None)

    for j in range(4):
        for k in range(8):
            rowidx[j, pl.ds(k * 16, 16)] = iota16 + (j * 128 + k * 16)

    for k in range(8):
        pltpu.sync_copy(zbuf, acc.at[pl.ds(s * ROWS_PER_TILE + k * ZROWS, ZROWS)])
    pltpu.sync_copy(histv.at[pl.ds(0, 32)], histsp.at[pl.ds(s * 32, 32)])
    plsc.subcore_barrier()

    ones16 = jnp.full((16,), 1.0, jnp.float32)
    ebase = s * EC

    def chunk(g, _):
        off = ebase + g * K
        pltpu.sync_copy(src_hbm.at[pl.ds(off, K)], srcbuf)
        pltpu.sync_copy(dst_hbm.at[pl.ds(off // 128, K // 128)], dstbuf)
        pltpu.async_copy(y_hbm.at[c].at[srcbuf], msg, sem).wait()
        for j in range(K // 128):
            pltpu.sync_copy(msg.at[pl.ds(j * 128, 128)],
                            acc.at[dstbuf.at[j]], add=True)

        def hist_group(v, _):
            sv = srcbuf[pl.ds(v * 16, 16)]
            r = jnp.right_shift(sv, 7)
            l = jnp.bitwise_and(sv, 127)
            plsc.addupdate_scatter(histv, [r, l], ones16)
            return _
        lax.fori_loop(0, K // 16, hist_group, None)
        return _
    lax.fori_loop(0, NCHUNK, chunk, None)

    plsc.subcore_barrier()

    pltpu.sync_copy(acc.at[pl.ds(s * ROWS_PER_TILE, ROWS_PER_TILE)],
                    a_hbm.at[c, pl.ds(s * ROWS_PER_TILE, ROWS_PER_TILE)])
    for j in range(4):
        pltpu.sync_copy(histv.at[pl.ds(j * 128, 128)],
                        histsp.at[rowidx.at[j]], add=True)
    plsc.subcore_barrier()
    pltpu.sync_copy(histsp.at[pl.ds(s * 32, 32)],
                    deg_hbm.at[c, pl.ds(s * 32, 32)])


def _sc_aggregate(y, srcp, dst2d):
    mesh = plsc.VectorSubcoreMesh(core_axis_name="c", subcore_axis_name="s")
    fn = pl.kernel(
        _sc_body,
        out_type=[
            jax.ShapeDtypeStruct((2, NP, H), jnp.float32),
            jax.ShapeDtypeStruct((2, 512, 128), jnp.float32),
        ],
        mesh=mesh,
        scratch_types=[
            pltpu.VMEM_SHARED((NP, H), jnp.float32),
            pltpu.VMEM_SHARED((512, 128), jnp.float32),
            pltpu.VMEM((K,), jnp.int32),
            pltpu.VMEM((K // 128, 128), jnp.int32),
            pltpu.VMEM((512, 128), jnp.float32),
            pltpu.VMEM((K, H), jnp.float32),
            pltpu.VMEM((4, 128), jnp.int32),
            pltpu.VMEM((ZROWS, H), jnp.float32),
            pltpu.SemaphoreType.DMA,
        ],
    )
    return fn(y, srcp, dst2d)


def _reduce_body(a_ref, d_ref, b1a_ref, b1b_ref, w2a_ref, w2b_ref, b2_ref,
                 out_ref, s0, s1):
    i = pl.program_id(0)

    @pl.when(i == 0)
    def _():
        s0[...] = jnp.zeros_like(s0)
        s1[...] = jnp.zeros_like(s1)

    w = (d_ref[0] + d_ref[1]) * 0.5
    node = i * 1024 + lax.broadcasted_iota(jnp.int32, (1, 1024), 1)
    w = jnp.where(node < N, w, 0.0)
    h0 = jnp.maximum(a_ref[0] + b1a_ref[...], 0.0)
    h1 = jnp.maximum(a_ref[1] + b1b_ref[...], 0.0)
    s0[...] += jnp.dot(w, h0, preferred_element_type=jnp.float32)
    s1[...] += jnp.dot(w, h1, preferred_element_type=jnp.float32)

    @pl.when(i == NP // 1024 - 1)
    def _():
        out = (jnp.dot(s0[...], w2a_ref[...], preferred_element_type=jnp.float32)
               + jnp.dot(s1[...], w2b_ref[...], preferred_element_type=jnp.float32))
        out_ref[...] = out * (1.0 / N) + b2_ref[...]


def _reduce(a, dr, b1a, b1b, w2a, w2b, b2):
    return pl.pallas_call(
        _reduce_body,
        grid=(NP // 1024,),
        in_specs=[
            pl.BlockSpec((2, 1024, H), lambda i: (0, i, 0)),
            pl.BlockSpec((2, 1, 1024), lambda i: (0, i, 0)),
            pl.BlockSpec((1, H), lambda i: (0, 0)),
            pl.BlockSpec((1, H), lambda i: (0, 0)),
            pl.BlockSpec((H, D), lambda i: (0, 0)),
            pl.BlockSpec((H, D), lambda i: (0, 0)),
            pl.BlockSpec((1, D), lambda i: (0, 0)),
        ],
        out_specs=pl.BlockSpec((1, D), lambda i: (0, 0)),
        out_shape=jax.ShapeDtypeStruct((1, D), jnp.float32),
        scratch_shapes=[
            pltpu.VMEM((1, H), jnp.float32),
            pltpu.VMEM((1, H), jnp.float32),
        ],
    )(a, dr, b1a, b1b, w2a, w2b, b2)


@jax.jit
def kernel(feats, edge_index, W1, b1, W2, b2):
    src = edge_index[0]
    dst = edge_index[1]
    xp = jnp.pad(feats, ((0, NP - N), (0, 0)))
    pad = jnp.full((EP - E,), NP - 1, jnp.int32)
    srcp = jnp.concatenate([src, pad])
    dst2d = jnp.concatenate([dst, pad]).reshape(EP // 128, 128)

    y = _project(xp, W1[:, :H], W1[:, H:])
    a, deg = _sc_aggregate(y, srcp, dst2d)
    dr = deg[:, :392, :].reshape(2, NP // 1024, 1024)

    return _reduce(a, dr, b1[:H].reshape(1, H), b1[H:].reshape(1, H),
                   W2[:H, :], W2[H:, :], b2.reshape(1, D))


# speedup vs baseline: 12.1603x; 12.1603x over previous
import functools

import jax
import jax.numpy as jnp
from jax import lax
from jax.experimental import pallas as pl
from jax.experimental.pallas import tpu as pltpu
from jax.experimental.pallas import tpu_sc as plsc

N = 50000
E = 800000
D = 64
H = 32
NP = 50176
EP = 802816
EC = 50176
K = 512
NCHUNK = EC // K
ROWS_PER_TILE = NP // 16
ZCOL = 392


def _proj_body(x_ref, w1a_ref, w1b_ref, y_ref):
    x = x_ref[...]
    y_ref[0] = jnp.dot(x, w1a_ref[...], preferred_element_type=jnp.float32)
    y_ref[1] = jnp.dot(x, w1b_ref[...], preferred_element_type=jnp.float32)


def _project(xp, w1a, w1b):
    return pl.pallas_call(
        _proj_body,
        grid=(NP // 1024,),
        in_specs=[
            pl.BlockSpec((1024, D), lambda i: (i, 0)),
            pl.BlockSpec((D, H), lambda i: (0, 0)),
            pl.BlockSpec((D, H), lambda i: (0, 0)),
        ],
        out_specs=pl.BlockSpec((2, 1024, H), lambda i: (0, i, 0)),
        out_shape=jax.ShapeDtypeStruct((2, NP, H), jnp.float32),
    )(xp, w1a, w1b)


def _sc_body(y_hbm, src_hbm, dst_hbm, a_hbm, deg_hbm,
             acc, histsp, srcbuf, dstbuf, msg, ones, zcol, sem):
    c = lax.axis_index("c")
    s = lax.axis_index("s")
    z16 = jnp.zeros((16,), jnp.float32)
    one16 = jnp.full((16,), 1.0, jnp.float32)

    for k in range(8):
        ones[pl.ds(k * 16, 16)] = one16
    def zero_zc(r, _):
        zcol[pl.ds(r * 16, 16)] = z16
        return _
    lax.fori_loop(0, ZCOL // 16, zero_zc, None)
    def zero_msg(r, _):
        msg[r, pl.ds(0, 16)] = z16
        msg[r, pl.ds(16, 16)] = z16
        return _
    lax.fori_loop(0, K, zero_msg, None)

    for k in range(6):
        pltpu.sync_copy(msg, acc.at[pl.ds(s * ROWS_PER_TILE + k * K, K)])
    pltpu.sync_copy(msg.at[pl.ds(0, ROWS_PER_TILE - 6 * K)],
                    acc.at[pl.ds(s * ROWS_PER_TILE + 6 * K,
                                 ROWS_PER_TILE - 6 * K)])
    for k in range(8):
        pltpu.sync_copy(zcol, histsp.at[pl.ds(s * ROWS_PER_TILE + k * ZCOL, ZCOL)])
    plsc.subcore_barrier()

    ebase = s * EC

    def chunk(g, _):
        row0 = pl.multiple_of(s * (EC // 128) + g * 8, 8)
        pltpu.sync_copy(src_hbm.at[pl.ds(row0, 8)], srcbuf)
        pltpu.sync_copy(dst_hbm.at[pl.ds(row0, 8)], dstbuf)
        for h in range(2):
            cps = [pltpu.async_copy(y_hbm.at[c].at[srcbuf.at[h * 4 + j]],
                                    msg.at[pl.ds(j * 128, 128)], sem)
                   for j in range(4)]
            for cp in cps:
                cp.wait()
            for j in range(4):
                pltpu.sync_copy(msg.at[pl.ds(j * 128, 128)],
                                acc.at[dstbuf.at[h * 4 + j]], add=True)

            @pl.when(c == h)
            def _():
                for j in range(4):
                    pltpu.sync_copy(ones, histsp.at[srcbuf.at[h * 4 + j]],
                                    add=True)
        return _
    lax.fori_loop(0, NCHUNK // 2, chunk, None)

    plsc.subcore_barrier()

    pltpu.sync_copy(acc.at[pl.ds(s * ROWS_PER_TILE, ROWS_PER_TILE)],
                    a_hbm.at[c, pl.ds(s * ROWS_PER_TILE, ROWS_PER_TILE)])
    pltpu.sync_copy(histsp.at[pl.ds(s * ROWS_PER_TILE, ROWS_PER_TILE)],
                    deg_hbm.at[c, pl.ds(s * ROWS_PER_TILE, ROWS_PER_TILE)])


def _sc_aggregate(y, src2d, dst2d):
    mesh = plsc.VectorSubcoreMesh(core_axis_name="c", subcore_axis_name="s")
    fn = pl.kernel(
        _sc_body,
        out_type=[
            jax.ShapeDtypeStruct((2, NP, H), jnp.float32),
            jax.ShapeDtypeStruct((2, NP), jnp.float32),
        ],
        mesh=mesh,
        compiler_params=pltpu.CompilerParams(
            needs_layout_passes=False, use_tc_tiling_on_sc=False),
        scratch_types=[
            pltpu.VMEM_SHARED((NP, H), jnp.float32),
            pltpu.VMEM_SHARED((NP,), jnp.float32),
            pltpu.VMEM((8, 128), jnp.int32),
            pltpu.VMEM((8, 128), jnp.int32),
            pltpu.VMEM((K, H), jnp.float32),
            pltpu.VMEM((128,), jnp.float32),
            pltpu.VMEM((ZCOL,), jnp.float32),
            pltpu.SemaphoreType.DMA,
        ],
    )
    return fn(y, src2d, dst2d)


def _reduce_body(a_ref, d_ref, b1a_ref, b1b_ref, w2a_ref, w2b_ref, b2_ref,
                 out_ref, s0, s1):
    i = pl.program_id(0)

    @pl.when(i == 0)
    def _():
        s0[...] = jnp.zeros_like(s0)
        s1[...] = jnp.zeros_like(s1)

    w = d_ref[0] + d_ref[1]
    node = i * 1024 + lax.broadcasted_iota(jnp.int32, (1024, 1), 0)
    w = jnp.where(node < N, w, 0.0)
    h0 = jnp.maximum(a_ref[0] + b1a_ref[...], 0.0)
    h1 = jnp.maximum(a_ref[1] + b1b_ref[...], 0.0)
    s0[...] += jnp.sum(h0 * w, axis=0, keepdims=True)
    s1[...] += jnp.sum(h1 * w, axis=0, keepdims=True)

    @pl.when(i == NP // 1024 - 1)
    def _():
        out = (jnp.dot(s0[...], w2a_ref[...], preferred_element_type=jnp.float32)
               + jnp.dot(s1[...], w2b_ref[...], preferred_element_type=jnp.float32))
        out_ref[...] = out * (1.0 / N) + b2_ref[...]


def _reduce(a, dr, b1a, b1b, w2a, w2b, b2):
    return pl.pallas_call(
        _reduce_body,
        grid=(NP // 1024,),
        in_specs=[
            pl.BlockSpec((2, 1024, H), lambda i: (0, i, 0)),
            pl.BlockSpec((2, 1024, 1), lambda i: (0, i, 0)),
            pl.BlockSpec((1, H), lambda i: (0, 0)),
            pl.BlockSpec((1, H), lambda i: (0, 0)),
            pl.BlockSpec((H, D), lambda i: (0, 0)),
            pl.BlockSpec((H, D), lambda i: (0, 0)),
            pl.BlockSpec((1, D), lambda i: (0, 0)),
        ],
        out_specs=pl.BlockSpec((1, D), lambda i: (0, 0)),
        out_shape=jax.ShapeDtypeStruct((1, D), jnp.float32),
        scratch_shapes=[
            pltpu.VMEM((1, H), jnp.float32),
            pltpu.VMEM((1, H), jnp.float32),
        ],
    )(a, dr, b1a, b1b, w2a, w2b, b2)


@jax.jit
def kernel(feats, edge_index, W1, b1, W2, b2):
    src = edge_index[0]
    dst = edge_index[1]
    xp = jnp.pad(feats, ((0, NP - N), (0, 0)))
    pad = jnp.full((EP - E,), NP - 1, jnp.int32)
    src2d = jnp.concatenate([src, pad]).reshape(EP // 128, 128)
    dst2d = jnp.concatenate([dst, pad]).reshape(EP // 128, 128)

    y = _project(xp, W1[:, :H], W1[:, H:])
    a, deg = _sc_aggregate(y, src2d, dst2d)
    dr = deg.reshape(2, NP, 1)

    return _reduce(a, dr, b1[:H].reshape(1, H), b1[H:].reshape(1, H),
                   W2[:H, :], W2[H:, :], b2.reshape(1, D))


# speedup vs baseline: 14.5755x; 1.1986x over previous
import jax
import jax.numpy as jnp
from jax import lax
from jax.experimental import pallas as pl
from jax.experimental.pallas import tpu as pltpu
from jax.experimental.pallas import tpu_sc as plsc

N = 50000
E = 800000
D = 64
H = 32
NP = 50176
EC = E // 16
K = 512
NFULL = EC // K
TAIL = EC - NFULL * K
PAD_CNT = float(16 * (K - TAIL))
ROWS_PER_TILE = NP // 16
ZCOL = 392


def _proj_body(x_ref, w1a_ref, w1b_ref, y_ref):
    x = x_ref[...]
    y_ref[0] = jnp.dot(x, w1a_ref[...], preferred_element_type=jnp.float32)
    y_ref[1] = jnp.dot(x, w1b_ref[...], preferred_element_type=jnp.float32)


def _project(xp, w1a, w1b):
    return pl.pallas_call(
        _proj_body,
        grid=(N // 2000,),
        in_specs=[
            pl.BlockSpec((2000, D), lambda i: (i, 0)),
            pl.BlockSpec((D, H), lambda i: (0, 0)),
            pl.BlockSpec((D, H), lambda i: (0, 0)),
        ],
        out_specs=pl.BlockSpec((2, 2000, H), lambda i: (0, i, 0)),
        out_shape=jax.ShapeDtypeStruct((2, NP, H), jnp.float32),
    )(xp, w1a, w1b)


def _sc_body(y_hbm, edge_hbm, a_hbm, deg_hbm,
             acc, histsp, srcbuf, dstbuf, msg, ones, zcol, sem):
    c = lax.axis_index("c")
    s = lax.axis_index("s")
    z16 = jnp.zeros((16,), jnp.float32)
    one16 = jnp.full((16,), 1.0, jnp.float32)

    for k in range(8):
        ones[pl.ds(k * 16, 16)] = one16

    def zero_zc(r, _):
        zcol[pl.ds(r * 16, 16)] = z16
        return _
    lax.fori_loop(0, ZCOL // 16, zero_zc, None)

    def zero_msg(r, _):
        msg[r, pl.ds(0, 16)] = z16
        msg[r, pl.ds(16, 16)] = z16
        return _
    lax.fori_loop(0, K, zero_msg, None)

    for k in range(6):
        pltpu.sync_copy(msg, acc.at[pl.ds(s * ROWS_PER_TILE + k * K, K)])
    pltpu.sync_copy(msg.at[pl.ds(0, ROWS_PER_TILE - 6 * K)],
                    acc.at[pl.ds(s * ROWS_PER_TILE + 6 * K,
                                 ROWS_PER_TILE - 6 * K)])
    for k in range(8):
        pltpu.sync_copy(zcol, histsp.at[pl.ds(s * ROWS_PER_TILE + k * ZCOL, ZCOL)])
    plsc.subcore_barrier()

    ebase = s * EC

    def do_chunk(hist_pred):
        cps = [pltpu.async_copy(y_hbm.at[c].at[srcbuf.at[pl.ds(j * 128, 128)]],
                                msg.at[pl.ds(j * 128, 128)], sem)
               for j in range(4)]
        for j in range(4):
            cps[j].wait()
            pltpu.sync_copy(msg.at[pl.ds(j * 128, 128)],
                            acc.at[dstbuf.at[pl.ds(j * 128, 128)]], add=True)

        @pl.when(hist_pred)
        def _():
            for j in range(4):
                pltpu.sync_copy(ones,
                                histsp.at[srcbuf.at[pl.ds(j * 128, 128)]],
                                add=True)

    def chunk(g, _):
        off = ebase + g * K
        pltpu.sync_copy(edge_hbm.at[0, pl.ds(off, K)], srcbuf)
        pltpu.sync_copy(edge_hbm.at[1, pl.ds(off, K)], dstbuf)
        do_chunk(lax.rem(g, 2) == c)
        return _
    lax.fori_loop(0, NFULL, chunk, None)

    def fill_tail(r, _):
        srcbuf[pl.ds(TAIL + r * 16, 16)] = jnp.zeros((16,), jnp.int32)
        dstbuf[pl.ds(TAIL + r * 16, 16)] = jnp.full((16,), N, jnp.int32)
        return _
    lax.fori_loop(0, (K - TAIL) // 16, fill_tail, None)
    toff = ebase + NFULL * K
    pltpu.sync_copy(edge_hbm.at[0, pl.ds(toff, TAIL)], srcbuf.at[pl.ds(0, TAIL)])
    pltpu.sync_copy(edge_hbm.at[1, pl.ds(toff, TAIL)], dstbuf.at[pl.ds(0, TAIL)])
    do_chunk(lax.rem(NFULL, 2) == c)

    plsc.subcore_barrier()

    pltpu.sync_copy(acc.at[pl.ds(s * ROWS_PER_TILE, ROWS_PER_TILE)],
                    a_hbm.at[c, pl.ds(s * ROWS_PER_TILE, ROWS_PER_TILE)])
    pltpu.sync_copy(histsp.at[pl.ds(s * ROWS_PER_TILE, ROWS_PER_TILE)],
                    deg_hbm.at[c, pl.ds(s * ROWS_PER_TILE, ROWS_PER_TILE)])


def _sc_aggregate(y, edge_index):
    mesh = plsc.VectorSubcoreMesh(core_axis_name="c", subcore_axis_name="s")
    fn = pl.kernel(
        _sc_body,
        out_type=[
            jax.ShapeDtypeStruct((2, NP, H), jnp.float32),
            jax.ShapeDtypeStruct((2, NP), jnp.float32),
        ],
        mesh=mesh,
        compiler_params=pltpu.CompilerParams(
            needs_layout_passes=False, use_tc_tiling_on_sc=False),
        scratch_types=[
            pltpu.VMEM_SHARED((NP, H), jnp.float32),
            pltpu.VMEM_SHARED((NP,), jnp.float32),
            pltpu.VMEM((K,), jnp.int32),
            pltpu.VMEM((K,), jnp.int32),
            pltpu.VMEM((K, H), jnp.float32),
            pltpu.VMEM((128,), jnp.float32),
            pltpu.VMEM((ZCOL,), jnp.float32),
            pltpu.SemaphoreType.DMA,
        ],
    )
    return fn(y, edge_index)


def _reduce_body(a_ref, d_ref, b1a_ref, b1b_ref, w2a_ref, w2b_ref, b2_ref,
                 out_ref, s0, s1):
    i = pl.program_id(0)

    @pl.when(i == 0)
    def _():
        s0[...] = jnp.zeros_like(s0)
        s1[...] = jnp.zeros_like(s1)

    w = d_ref[0:1, :] + d_ref[1:2, :]
    node = i * 1024 + lax.broadcasted_iota(jnp.int32, (1, 1024), 1)
    w = jnp.where(node < N, w, 0.0)
    w = w - jnp.where(node == 0, PAD_CNT, 0.0)
    h0 = jnp.maximum(a_ref[0] + b1a_ref[...], 0.0)
    h1 = jnp.maximum(a_ref[1] + b1b_ref[...], 0.0)
    s0[...] += jnp.dot(w, h0, preferred_element_type=jnp.float32)
    s1[...] += jnp.dot(w, h1, preferred_element_type=jnp.float32)

    @pl.when(i == NP // 1024 - 1)
    def _():
        out = (jnp.dot(s0[...], w2a_ref[...], preferred_element_type=jnp.float32)
               + jnp.dot(s1[...], w2b_ref[...], preferred_element_type=jnp.float32))
        out_ref[...] = out * (1.0 / N) + b2_ref[...]


def _reduce(a, deg, b1a, b1b, w2a, w2b, b2):
    return pl.pallas_call(
        _reduce_body,
        grid=(NP // 1024,),
        in_specs=[
            pl.BlockSpec((2, 1024, H), lambda i: (0, i, 0)),
            pl.BlockSpec((2, 1024), lambda i: (0, i)),
            pl.BlockSpec((1, H), lambda i: (0, 0)),
            pl.BlockSpec((1, H), lambda i: (0, 0)),
            pl.BlockSpec((H, D), lambda i: (0, 0)),
            pl.BlockSpec((H, D), lambda i: (0, 0)),
            pl.BlockSpec((1, D), lambda i: (0, 0)),
        ],
        out_specs=pl.BlockSpec((1, D), lambda i: (0, 0)),
        out_shape=jax.ShapeDtypeStruct((1, D), jnp.float32),
        scratch_shapes=[
            pltpu.VMEM((1, H), jnp.float32),
            pltpu.VMEM((1, H), jnp.float32),
        ],
    )(a, deg, b1a, b1b, w2a, w2b, b2)


@jax.jit
def kernel(feats, edge_index, W1, b1, W2, b2):
    y = _project(feats, W1[:, :H], W1[:, H:])
    a, deg = _sc_aggregate(y, edge_index)
    return _reduce(a, deg, b1[:H].reshape(1, H), b1[H:].reshape(1, H),
                   W2[:H, :], W2[H:, :], b2.reshape(1, D))


# speedup vs baseline: 17.1077x; 1.1737x over previous
import jax
import jax.numpy as jnp
from jax import lax
from jax.experimental import pallas as pl
from jax.experimental.pallas import tpu as pltpu
from jax.experimental.pallas import tpu_sc as plsc

N = 50000
E = 800000
D = 64
H = 32
NP = 50176
EC = E // 16
K = 512
B = 4096
NBLK = EC // B
REM = EC - NBLK * B
TAIL = REM - K
PAD_CNT = float(16 * (K - TAIL))
ROWS_PER_TILE = NP // 16
ZCOL = 392


def _proj_body(x_ref, w1a_ref, w1b_ref, y_ref):
    x = x_ref[...]
    y_ref[0] = jnp.dot(x, w1a_ref[...], preferred_element_type=jnp.float32)
    y_ref[1] = jnp.dot(x, w1b_ref[...], preferred_element_type=jnp.float32)


def _project(xp, w1a, w1b):
    return pl.pallas_call(
        _proj_body,
        grid=(N // 2000,),
        in_specs=[
            pl.BlockSpec((2000, D), lambda i: (i, 0)),
            pl.BlockSpec((D, H), lambda i: (0, 0)),
            pl.BlockSpec((D, H), lambda i: (0, 0)),
        ],
        out_specs=pl.BlockSpec((2, 2000, H), lambda i: (0, i, 0)),
        out_shape=jax.ShapeDtypeStruct((2, NP, H), jnp.float32),
    )(xp, w1a, w1b)


def _sc_body(y_hbm, edge_hbm, a_hbm, deg_hbm,
             acc, histsp, srcbuf, dstbuf, msg, ones, zcol, sem):
    c = lax.axis_index("c")
    s = lax.axis_index("s")
    z16 = jnp.zeros((16,), jnp.float32)
    one16 = jnp.full((16,), 1.0, jnp.float32)

    for k in range(8):
        ones[pl.ds(k * 16, 16)] = one16

    def zero_zc(r, _):
        zcol[pl.ds(r * 16, 16)] = z16
        return _
    lax.fori_loop(0, ZCOL // 16, zero_zc, None)

    def zero_msg(r, _):
        msg[r, pl.ds(0, 16)] = z16
        msg[r, pl.ds(16, 16)] = z16
        return _
    lax.fori_loop(0, K, zero_msg, None)

    for k in range(6):
        pltpu.sync_copy(msg, acc.at[pl.ds(s * ROWS_PER_TILE + k * K, K)])
    pltpu.sync_copy(msg.at[pl.ds(0, ROWS_PER_TILE - 6 * K)],
                    acc.at[pl.ds(s * ROWS_PER_TILE + 6 * K,
                                 ROWS_PER_TILE - 6 * K)])
    for k in range(8):
        pltpu.sync_copy(zcol, histsp.at[pl.ds(s * ROWS_PER_TILE + k * ZCOL, ZCOL)])
    plsc.subcore_barrier()

    ebase = s * EC

    def do_chunk(u, hist_static_parity):
        base = u * K
        cps = [pltpu.async_copy(
                   y_hbm.at[c].at[srcbuf.at[pl.ds(base + j * 128, 128)]],
                   msg.at[pl.ds(j * 128, 128)], sem)
               for j in range(4)]
        for j in range(4):
            cps[j].wait()
            pltpu.sync_copy(msg.at[pl.ds(j * 128, 128)],
                            acc.at[dstbuf.at[pl.ds(base + j * 128, 128)]],
                            add=True)

        @pl.when(c == hist_static_parity)
        def _():
            for j in range(4):
                pltpu.sync_copy(
                    ones, histsp.at[srcbuf.at[pl.ds(base + j * 128, 128)]],
                    add=True)

    def block(b, _):
        off = ebase + b * B
        pltpu.sync_copy(edge_hbm.at[0, pl.ds(off, B)], srcbuf)
        pltpu.sync_copy(edge_hbm.at[1, pl.ds(off, B)], dstbuf)
        for u in range(B // K):
            do_chunk(u, u % 2)
        return _
    lax.fori_loop(0, NBLK, block, None)

    def fill_tail(r, _):
        srcbuf[pl.ds(REM + r * 16, 16)] = jnp.zeros((16,), jnp.int32)
        dstbuf[pl.ds(REM + r * 16, 16)] = jnp.full((16,), N, jnp.int32)
        return _
    lax.fori_loop(0, (2 * K - REM) // 16, fill_tail, None)
    roff = ebase + NBLK * B
    pltpu.sync_copy(edge_hbm.at[0, pl.ds(roff, REM)], srcbuf.at[pl.ds(0, REM)])
    pltpu.sync_copy(edge_hbm.at[1, pl.ds(roff, REM)], dstbuf.at[pl.ds(0, REM)])
    do_chunk(0, 0)
    do_chunk(1, 1)

    plsc.subcore_barrier()

    pltpu.sync_copy(acc.at[pl.ds(s * ROWS_PER_TILE, ROWS_PER_TILE)],
                    a_hbm.at[c, pl.ds(s * ROWS_PER_TILE, ROWS_PER_TILE)])
    pltpu.sync_copy(histsp.at[pl.ds(s * ROWS_PER_TILE, ROWS_PER_TILE)],
                    deg_hbm.at[c, pl.ds(s * ROWS_PER_TILE, ROWS_PER_TILE)])


def _sc_aggregate(y, edge_index):
    mesh = plsc.VectorSubcoreMesh(core_axis_name="c", subcore_axis_name="s")
    fn = pl.kernel(
        _sc_body,
        out_type=[
            jax.ShapeDtypeStruct((2, NP, H), jnp.float32),
            jax.ShapeDtypeStruct((2, NP), jnp.float32),
        ],
        mesh=mesh,
        compiler_params=pltpu.CompilerParams(
            needs_layout_passes=False, use_tc_tiling_on_sc=False),
        scratch_types=[
            pltpu.VMEM_SHARED((NP, H), jnp.float32),
            pltpu.VMEM_SHARED((NP,), jnp.float32),
            pltpu.VMEM((B,), jnp.int32),
            pltpu.VMEM((B,), jnp.int32),
            pltpu.VMEM((K, H), jnp.float32),
            pltpu.VMEM((128,), jnp.float32),
            pltpu.VMEM((ZCOL,), jnp.float32),
            pltpu.SemaphoreType.DMA,
        ],
    )
    return fn(y, edge_index)


def _reduce_body(a_ref, d_ref, b1a_ref, b1b_ref, w2a_ref, w2b_ref, b2_ref,
                 out_ref, s0, s1):
    i = pl.program_id(0)

    @pl.when(i == 0)
    def _():
        s0[...] = jnp.zeros_like(s0)
        s1[...] = jnp.zeros_like(s1)

    w = d_ref[0:1, :] + d_ref[1:2, :]
    node = i * 1024 + lax.broadcasted_iota(jnp.int32, (1, 1024), 1)
    w = jnp.where(node < N, w, 0.0)
    w = w - jnp.where(node == 0, PAD_CNT, 0.0)
    h0 = jnp.maximum(a_ref[0] + b1a_ref[...], 0.0)
    h1 = jnp.maximum(a_ref[1] + b1b_ref[...], 0.0)
    s0[...] += jnp.dot(w, h0, preferred_element_type=jnp.float32)
    s1[...] += jnp.dot(w, h1, preferred_element_type=jnp.float32)

    @pl.when(i == NP // 1024 - 1)
    def _():
        out = (jnp.dot(s0[...], w2a_ref[...], preferred_element_type=jnp.float32)
               + jnp.dot(s1[...], w2b_ref[...], preferred_element_type=jnp.float32))
        out_ref[...] = out * (1.0 / N) + b2_ref[...]


def _reduce(a, deg, b1a, b1b, w2a, w2b, b2):
    return pl.pallas_call(
        _reduce_body,
        grid=(NP // 1024,),
        in_specs=[
            pl.BlockSpec((2, 1024, H), lambda i: (0, i, 0)),
            pl.BlockSpec((2, 1024), lambda i: (0, i)),
            pl.BlockSpec((1, H), lambda i: (0, 0)),
            pl.BlockSpec((1, H), lambda i: (0, 0)),
            pl.BlockSpec((H, D), lambda i: (0, 0)),
            pl.BlockSpec((H, D), lambda i: (0, 0)),
            pl.BlockSpec((1, D), lambda i: (0, 0)),
        ],
        out_specs=pl.BlockSpec((1, D), lambda i: (0, 0)),
        out_shape=jax.ShapeDtypeStruct((1, D), jnp.float32),
        scratch_shapes=[
            pltpu.VMEM((1, H), jnp.float32),
            pltpu.VMEM((1, H), jnp.float32),
        ],
    )(a, deg, b1a, b1b, w2a, w2b, b2)


@jax.jit
def kernel(feats, edge_index, W1, b1, W2, b2):
    y = _project(feats, W1[:, :H], W1[:, H:])
    a, deg = _sc_aggregate(y, edge_index)
    return _reduce(a, deg, b1[:H].reshape(1, H), b1[H:].reshape(1, H),
                   W2[:H, :], W2[H:, :], b2.reshape(1, D))


# speedup vs baseline: 18.6989x; 1.0930x over previous
import jax
import jax.numpy as jnp
from jax import lax
from jax.experimental import pallas as pl
from jax.experimental.pallas import tpu as pltpu
from jax.experimental.pallas import tpu_sc as plsc

N = 50000
E = 800000
D = 64
H = 32
NP = 50176
EC = E // 16
K = 512
B = 4096
NBLK = EC // B
REM = EC - NBLK * B
TAIL = REM - K
PAD_CNT = float(16 * (K - TAIL))
ROWS_PER_TILE = NP // 16
ZCOL = 392


def _proj_body(x_ref, w1a_ref, w1b_ref, y_ref):
    x = x_ref[...]
    y_ref[0] = jnp.dot(x, w1a_ref[...], preferred_element_type=jnp.float32)
    y_ref[1] = jnp.dot(x, w1b_ref[...], preferred_element_type=jnp.float32)


def _project(xp, w1a, w1b):
    return pl.pallas_call(
        _proj_body,
        grid=(N // 2000,),
        in_specs=[
            pl.BlockSpec((2000, D), lambda i: (i, 0)),
            pl.BlockSpec((D, H), lambda i: (0, 0)),
            pl.BlockSpec((D, H), lambda i: (0, 0)),
        ],
        out_specs=pl.BlockSpec((2, 2000, H), lambda i: (0, i, 0)),
        out_shape=jax.ShapeDtypeStruct((2, NP, H), jnp.float32),
    )(xp, w1a, w1b)


def _sc_body(y_hbm, edge_hbm, a_hbm, deg_hbm,
             acc, histsp, srcbuf, dstbuf, msg, ones, zcol, semg, sems):
    c = lax.axis_index("c")
    s = lax.axis_index("s")
    z16 = jnp.zeros((16,), jnp.float32)
    one16 = jnp.full((16,), 1.0, jnp.float32)

    for k in range(8):
        ones[pl.ds(k * 16, 16)] = one16

    def zero_zc(r, _):
        zcol[pl.ds(r * 16, 16)] = z16
        return _
    lax.fori_loop(0, ZCOL // 16, zero_zc, None)

    def zero_msg(r, _):
        msg[r, pl.ds(0, 16)] = z16
        msg[r, pl.ds(16, 16)] = z16
        return _
    lax.fori_loop(0, K, zero_msg, None)

    for k in range(6):
        pltpu.sync_copy(msg, acc.at[pl.ds(s * ROWS_PER_TILE + k * K, K)])
    pltpu.sync_copy(msg.at[pl.ds(0, ROWS_PER_TILE - 6 * K)],
                    acc.at[pl.ds(s * ROWS_PER_TILE + 6 * K,
                                 ROWS_PER_TILE - 6 * K)])
    for k in range(8):
        pltpu.sync_copy(zcol, histsp.at[pl.ds(s * ROWS_PER_TILE + k * ZCOL, ZCOL)])
    plsc.subcore_barrier()

    ebase = s * EC
    CK = 256
    UPB = B // CK

    def issue_gathers(u):
        slot = u % 2
        return [pltpu.async_copy(
                    y_hbm.at[c].at[srcbuf.at[pl.ds(u * CK + j * 128, 128)]],
                    msg.at[pl.ds(slot * CK + j * 128, 128)], semg[slot])
                for j in range(2)]

    def issue_scatters(u):
        slot = u % 2
        return [pltpu.async_copy(
                    msg.at[pl.ds(slot * CK + j * 128, 128)],
                    acc.at[dstbuf.at[pl.ds(u * CK + j * 128, 128)]],
                    sems[slot], add=True)
                for j in range(2)]

    def do_hist(u):
        @pl.when(c == u % 2)
        def _():
            for j in range(2):
                pltpu.sync_copy(
                    ones, histsp.at[srcbuf.at[pl.ds(u * CK + j * 128, 128)]],
                    add=True)

    def block(b, _):
        off = ebase + b * B
        pltpu.sync_copy(edge_hbm.at[0, pl.ds(off, B)], srcbuf)
        pltpu.sync_copy(edge_hbm.at[1, pl.ds(off, B)], dstbuf)
        g_cps = issue_gathers(0)
        s_cps = [None, None]
        for u in range(UPB):
            if u + 1 < UPB:
                if s_cps[(u + 1) % 2] is not None:
                    for cp in s_cps[(u + 1) % 2]:
                        cp.wait()
                ng = issue_gathers(u + 1)
            for cp in g_cps:
                cp.wait()
            s_cps[u % 2] = issue_scatters(u)
            do_hist(u)
            if u + 1 < UPB:
                g_cps = ng
        for slot in range(2):
            for cp in s_cps[slot]:
                cp.wait()
        return _
    lax.fori_loop(0, NBLK, block, None)

    def fill_tail(r, _):
        srcbuf[pl.ds(REM + r * 16, 16)] = jnp.zeros((16,), jnp.int32)
        dstbuf[pl.ds(REM + r * 16, 16)] = jnp.full((16,), N, jnp.int32)
        return _
    lax.fori_loop(0, (4 * CK - REM) // 16, fill_tail, None)
    roff = ebase + NBLK * B
    pltpu.sync_copy(edge_hbm.at[0, pl.ds(roff, REM)], srcbuf.at[pl.ds(0, REM)])
    pltpu.sync_copy(edge_hbm.at[1, pl.ds(roff, REM)], dstbuf.at[pl.ds(0, REM)])
    for u in range(4):
        for cp in issue_gathers(u):
            cp.wait()
        for cp in issue_scatters(u):
            cp.wait()
        do_hist(u)

    plsc.subcore_barrier()

    pltpu.sync_copy(acc.at[pl.ds(s * ROWS_PER_TILE, ROWS_PER_TILE)],
                    a_hbm.at[c, pl.ds(s * ROWS_PER_TILE, ROWS_PER_TILE)])
    pltpu.sync_copy(histsp.at[pl.ds(s * ROWS_PER_TILE, ROWS_PER_TILE)],
                    deg_hbm.at[c, pl.ds(s * ROWS_PER_TILE, ROWS_PER_TILE)])


def _sc_aggregate(y, edge_index):
    mesh = plsc.VectorSubcoreMesh(core_axis_name="c", subcore_axis_name="s")
    fn = pl.kernel(
        _sc_body,
        out_type=[
            jax.ShapeDtypeStruct((2, NP, H), jnp.float32),
            jax.ShapeDtypeStruct((2, NP), jnp.float32),
        ],
        mesh=mesh,
        compiler_params=pltpu.CompilerParams(
            needs_layout_passes=False, use_tc_tiling_on_sc=False),
        scratch_types=[
            pltpu.VMEM_SHARED((NP, H), jnp.float32),
            pltpu.VMEM_SHARED((NP,), jnp.float32),
            pltpu.VMEM((B,), jnp.int32),
            pltpu.VMEM((B,), jnp.int32),
            pltpu.VMEM((K, H), jnp.float32),
            pltpu.VMEM((128,), jnp.float32),
            pltpu.VMEM((ZCOL,), jnp.float32),
            [pltpu.SemaphoreType.DMA] * 2,
            [pltpu.SemaphoreType.DMA] * 2,
        ],
    )
    return fn(y, edge_index)


def _reduce_body(a_ref, d_ref, b1a_ref, b1b_ref, w2a_ref, w2b_ref, b2_ref,
                 out_ref, s0, s1):
    i = pl.program_id(0)

    @pl.when(i == 0)
    def _():
        s0[...] = jnp.zeros_like(s0)
        s1[...] = jnp.zeros_like(s1)

    w = d_ref[0:1, :] + d_ref[1:2, :]
    node = i * 1024 + lax.broadcasted_iota(jnp.int32, (1, 1024), 1)
    w = jnp.where(node < N, w, 0.0)
    w = w - jnp.where(node == 0, PAD_CNT, 0.0)
    h0 = jnp.maximum(a_ref[0] + b1a_ref[...], 0.0)
    h1 = jnp.maximum(a_ref[1] + b1b_ref[...], 0.0)
    s0[...] += jnp.dot(w, h0, preferred_element_type=jnp.float32)
    s1[...] += jnp.dot(w, h1, preferred_element_type=jnp.float32)

    @pl.when(i == NP // 1024 - 1)
    def _():
        out = (jnp.dot(s0[...], w2a_ref[...], preferred_element_type=jnp.float32)
               + jnp.dot(s1[...], w2b_ref[...], preferred_element_type=jnp.float32))
        out_ref[...] = out * (1.0 / N) + b2_ref[...]


def _reduce(a, deg, b1a, b1b, w2a, w2b, b2):
    return pl.pallas_call(
        _reduce_body,
        grid=(NP // 1024,),
        in_specs=[
            pl.BlockSpec((2, 1024, H), lambda i: (0, i, 0)),
            pl.BlockSpec((2, 1024), lambda i: (0, i)),
            pl.BlockSpec((1, H), lambda i: (0, 0)),
            pl.BlockSpec((1, H), lambda i: (0, 0)),
            pl.BlockSpec((H, D), lambda i: (0, 0)),
            pl.BlockSpec((H, D), lambda i: (0, 0)),
            pl.BlockSpec((1, D), lambda i: (0, 0)),
        ],
        out_specs=pl.BlockSpec((1, D), lambda i: (0, 0)),
        out_shape=jax.ShapeDtypeStruct((1, D), jnp.float32),
        scratch_shapes=[
            pltpu.VMEM((1, H), jnp.float32),
            pltpu.VMEM((1, H), jnp.float32),
        ],
    )(a, deg, b1a, b1b, w2a, w2b, b2)


@jax.jit
def kernel(feats, edge_index, W1, b1, W2, b2):
    y = _project(feats, W1[:, :H], W1[:, H:])
    a, deg = _sc_aggregate(y, edge_index)
    return _reduce(a, deg, b1[:H].reshape(1, H), b1[H:].reshape(1, H),
                   W2[:H, :], W2[H:, :], b2.reshape(1, D))


# speedup vs baseline: 20.9476x; 1.1203x over previous
import jax
import jax.numpy as jnp
from jax import lax
from jax.experimental import pallas as pl
from jax.experimental.pallas import tpu as pltpu
from jax.experimental.pallas import tpu_sc as plsc

N = 50000
E = 800000
D = 64
H = 32
NP = 50176
EC = E // 16
K = 512
B = 2048
NBLK = EC // B
REM = EC - NBLK * B
TAIL = REM - K
PAD_CNT = float(16 * (K - TAIL))
ROWS_PER_TILE = NP // 16
ZCOL = 392


def _proj_body(x_ref, w1a_ref, w1b_ref, y_ref):
    x = x_ref[...]
    y_ref[0] = jnp.dot(x, w1a_ref[...], preferred_element_type=jnp.float32)
    y_ref[1] = jnp.dot(x, w1b_ref[...], preferred_element_type=jnp.float32)


def _project(xp, w1a, w1b):
    return pl.pallas_call(
        _proj_body,
        grid=(N // 2000,),
        in_specs=[
            pl.BlockSpec((2000, D), lambda i: (i, 0)),
            pl.BlockSpec((D, H), lambda i: (0, 0)),
            pl.BlockSpec((D, H), lambda i: (0, 0)),
        ],
        out_specs=pl.BlockSpec((2, 2000, H), lambda i: (0, i, 0)),
        out_shape=jax.ShapeDtypeStruct((2, NP, H), jnp.float32),
    )(xp, w1a, w1b)


def _sc_body(y_hbm, edge_hbm, b1_hbm, s_hbm,
             acc, histsp, srcbuf, dstbuf, msg, ones, degbuf, b1buf,
             semg, sems):
    c = lax.axis_index("c")
    s = lax.axis_index("s")
    z16 = jnp.zeros((16,), jnp.float32)
    one16 = jnp.full((16,), 1.0, jnp.float32)

    for k in range(8):
        ones[pl.ds(k * 16, 16)] = one16

    def zero_deg(r, _):
        degbuf[pl.ds(r * 16, 16)] = z16
        return _
    lax.fori_loop(0, ROWS_PER_TILE // 16, zero_deg, None)

    def zero_msg(r, _):
        msg[r, pl.ds(0, 16)] = z16
        msg[r, pl.ds(16, 16)] = z16
        return _
    lax.fori_loop(0, K, zero_msg, None)

    for k in range(6):
        pltpu.sync_copy(msg, acc.at[pl.ds(s * ROWS_PER_TILE + k * K, K)])
    pltpu.sync_copy(msg.at[pl.ds(0, ROWS_PER_TILE - 6 * K)],
                    acc.at[pl.ds(s * ROWS_PER_TILE + 6 * K,
                                 ROWS_PER_TILE - 6 * K)])
    pltpu.sync_copy(degbuf, histsp.at[pl.ds(s * ROWS_PER_TILE, ROWS_PER_TILE)])
    plsc.subcore_barrier()

    ebase = s * EC
    CK = 256
    UPB = B // CK

    def issue_gathers(u):
        slot = u % 2
        return [pltpu.async_copy(
                    y_hbm.at[c].at[srcbuf.at[pl.ds(u * CK + j * 128, 128)]],
                    msg.at[pl.ds(slot * CK + j * 128, 128)], semg[slot])
                for j in range(2)]

    def issue_scatters(u):
        slot = u % 2
        return [pltpu.async_copy(
                    msg.at[pl.ds(slot * CK + j * 128, 128)],
                    acc.at[dstbuf.at[pl.ds(u * CK + j * 128, 128)]],
                    sems[slot], add=True)
                for j in range(2)]

    def do_hist(u):
        for j in range(2):
            pltpu.sync_copy(
                ones, histsp.at[srcbuf.at[pl.ds(u * CK + j * 128, 128)]],
                add=True)

    def block(b, _):
        off = ebase + b * B
        pltpu.sync_copy(edge_hbm.at[0, pl.ds(off, B)], srcbuf)
        pltpu.sync_copy(edge_hbm.at[1, pl.ds(off, B)], dstbuf)
        g_cps = issue_gathers(0)
        s_cps = [None, None]
        for u in range(UPB):
            if u + 1 < UPB:
                if s_cps[(u + 1) % 2] is not None:
                    for cp in s_cps[(u + 1) % 2]:
                        cp.wait()
                ng = issue_gathers(u + 1)
            for cp in g_cps:
                cp.wait()
            s_cps[u % 2] = issue_scatters(u)
            do_hist(u)
            if u + 1 < UPB:
                g_cps = ng
        for slot in range(2):
            for cp in s_cps[slot]:
                cp.wait()
        return _
    lax.fori_loop(0, NBLK, block, None)

    def fill_tail(r, _):
        srcbuf[pl.ds(REM + r * 16, 16)] = jnp.zeros((16,), jnp.int32)
        dstbuf[pl.ds(REM + r * 16, 16)] = jnp.full((16,), N, jnp.int32)
        return _
    lax.fori_loop(0, (4 * CK - REM) // 16, fill_tail, None)
    roff = ebase + NBLK * B
    pltpu.sync_copy(edge_hbm.at[0, pl.ds(roff, REM)], srcbuf.at[pl.ds(0, REM)])
    pltpu.sync_copy(edge_hbm.at[1, pl.ds(roff, REM)], dstbuf.at[pl.ds(0, REM)])
    for u in range(4):
        for cp in issue_gathers(u):
            cp.wait()
        for cp in issue_scatters(u):
            cp.wait()
        do_hist(u)

    plsc.subcore_barrier()

    pltpu.sync_copy(histsp.at[pl.ds(s * ROWS_PER_TILE, ROWS_PER_TILE)], degbuf)
    pltpu.sync_copy(b1_hbm.at[c], b1buf)

    @pl.when(s == 0)
    def _():
        v = degbuf[pl.ds(0, 16)]
        lane = lax.iota(jnp.int32, 16)
        degbuf[pl.ds(0, 16)] = v - jnp.where(lane == 0, PAD_CNT, 0.0)

    b1lo = b1buf[pl.ds(0, 16)]
    b1hi = b1buf[pl.ds(16, 16)]
    nrows = jnp.minimum(ROWS_PER_TILE, N - s * ROWS_PER_TILE)

    def weigh_rows(carry_chunk):
        k, nch = carry_chunk
        rcount = jnp.clip(nrows - k * 512, 0, nch)

        def row(r, sacc):
            a0, a1 = sacc
            dj = plsc.load_gather(degbuf, [jnp.full((16,), k * 512 + r,
                                                    jnp.int32)])
            m0 = msg[r, pl.ds(0, 16)]
            m1 = msg[r, pl.ds(16, 16)]
            a0 = a0 + dj * jnp.maximum(m0 + b1lo, 0.0)
            a1 = a1 + dj * jnp.maximum(m1 + b1hi, 0.0)
            return (a0, a1)
        return rcount, row

    acc0 = jnp.zeros((16,), jnp.float32)
    acc1 = jnp.zeros((16,), jnp.float32)
    for k in range(7):
        nch = 512 if k < 6 else 64
        pltpu.sync_copy(acc.at[pl.ds(s * ROWS_PER_TILE + k * 512, nch)],
                        msg.at[pl.ds(0, nch)])
        rcount, row = weigh_rows((k, nch))
        acc0, acc1 = lax.fori_loop(0, rcount, row, (acc0, acc1))

    ones[pl.ds(0, 16)] = acc0
    ones[pl.ds(16, 16)] = acc1
    pltpu.sync_copy(ones.at[pl.ds(0, 32)], s_hbm.at[c, s])


def _sc_aggregate(y, edge_index, b1):
    mesh = plsc.VectorSubcoreMesh(core_axis_name="c", subcore_axis_name="s")
    fn = pl.kernel(
        _sc_body,
        out_type=jax.ShapeDtypeStruct((2, 16, H), jnp.float32),
        mesh=mesh,
        compiler_params=pltpu.CompilerParams(
            needs_layout_passes=False, use_tc_tiling_on_sc=False),
        scratch_types=[
            pltpu.VMEM_SHARED((NP, H), jnp.float32),
            pltpu.VMEM_SHARED((NP,), jnp.float32),
            pltpu.VMEM((B,), jnp.int32),
            pltpu.VMEM((B,), jnp.int32),
            pltpu.VMEM((K, H), jnp.float32),
            pltpu.VMEM((128,), jnp.float32),
            pltpu.VMEM((ROWS_PER_TILE,), jnp.float32),
            pltpu.VMEM((H,), jnp.float32),
            [pltpu.SemaphoreType.DMA] * 2,
            [pltpu.SemaphoreType.DMA] * 2,
        ],
    )
    return fn(y, edge_index, b1)


def _finish_body(s_ref, w2a_ref, w2b_ref, b2_ref, out_ref):
    s0 = jnp.sum(s_ref[0], axis=0, keepdims=True)
    s1 = jnp.sum(s_ref[1], axis=0, keepdims=True)
    out = (jnp.dot(s0, w2a_ref[...], preferred_element_type=jnp.float32)
           + jnp.dot(s1, w2b_ref[...], preferred_element_type=jnp.float32))
    out_ref[...] = out * (1.0 / N) + b2_ref[...]


def _finish(sp, w2a, w2b, b2):
    return pl.pallas_call(
        _finish_body,
        out_shape=jax.ShapeDtypeStruct((1, D), jnp.float32),
    )(sp, w2a, w2b, b2)


@jax.jit
def kernel(feats, edge_index, W1, b1, W2, b2):
    y = _project(feats, W1[:, :H], W1[:, H:])
    sp = _sc_aggregate(y, edge_index, b1.reshape(2, H))
    return _finish(sp, W2[:H, :], W2[H:, :], b2.reshape(1, D))


# speedup vs baseline: 22.4381x; 1.0712x over previous
import jax
import jax.numpy as jnp
from jax import lax
from jax.experimental import pallas as pl
from jax.experimental.pallas import tpu as pltpu
from jax.experimental.pallas import tpu_sc as plsc

N = 50000
E = 800000
D = 64
H = 32
NP = 50176
EC = E // 16
K = 512
B = 2048
NBLK = EC // B
REM = EC - NBLK * B
TAIL = REM - K
PAD_CNT = float(16 * (K - TAIL))
ROWS_PER_TILE = NP // 16
ZCOL = 392


def _proj_body(x_ref, bd0_ref, bd1_ref, y_ref):
    x = x_ref[...]
    y_ref[0] = jnp.dot(x, bd0_ref[...], preferred_element_type=jnp.float32)
    y_ref[1] = jnp.dot(x, bd1_ref[...], preferred_element_type=jnp.float32)


def _project(xp, bd0, bd1):
    return pl.pallas_call(
        _proj_body,
        grid=(NP // 1024,),
        in_specs=[
            pl.BlockSpec((256, 4 * D), lambda i: (i, 0)),
            pl.BlockSpec((4 * D, 128), lambda i: (0, 0)),
            pl.BlockSpec((4 * D, 128), lambda i: (0, 0)),
        ],
        out_specs=pl.BlockSpec((2, 256, 128), lambda i: (0, i, 0)),
        out_shape=jax.ShapeDtypeStruct((2, NP * H // 128, 128), jnp.float32),
    )(xp, bd0, bd1)


def _sc_body(y_hbm, edge_hbm, b1_hbm, s_hbm,
             acc, histsp, srcbuf, dstbuf, msg, ones, degbuf, b1buf,
             semg, sems):
    c = lax.axis_index("c")
    s = lax.axis_index("s")
    z16 = jnp.zeros((16,), jnp.float32)
    one16 = jnp.full((16,), 1.0, jnp.float32)

    for k in range(8):
        ones[pl.ds(k * 16, 16)] = one16

    def zero_deg(r, _):
        degbuf[pl.ds(r * 16, 16)] = z16
        return _
    lax.fori_loop(0, ROWS_PER_TILE // 16, zero_deg, None)

    def zero_msg(r, _):
        msg[r, pl.ds(0, 16)] = z16
        msg[r, pl.ds(16, 16)] = z16
        return _
    lax.fori_loop(0, K, zero_msg, None)

    for k in range(6):
        pltpu.sync_copy(msg, acc.at[pl.ds(s * ROWS_PER_TILE + k * K, K)])
    pltpu.sync_copy(msg.at[pl.ds(0, ROWS_PER_TILE - 6 * K)],
                    acc.at[pl.ds(s * ROWS_PER_TILE + 6 * K,
                                 ROWS_PER_TILE - 6 * K)])
    pltpu.sync_copy(degbuf, histsp.at[pl.ds(s * ROWS_PER_TILE, ROWS_PER_TILE)])
    plsc.subcore_barrier()

    ebase = s * EC
    CK = 256
    UPB = B // CK

    def issue_gathers(u):
        slot = u % 2
        return [pltpu.async_copy(
                    y_hbm.at[c].at[srcbuf.at[pl.ds(u * CK + j * 128, 128)]],
                    msg.at[pl.ds(slot * CK + j * 128, 128)], semg[slot])
                for j in range(2)]

    def issue_scatters(u):
        slot = u % 2
        return [pltpu.async_copy(
                    msg.at[pl.ds(slot * CK + j * 128, 128)],
                    acc.at[dstbuf.at[pl.ds(u * CK + j * 128, 128)]],
                    sems[slot], add=True)
                for j in range(2)]

    def do_hist(u):
        for j in range(2):
            pltpu.sync_copy(
                ones, histsp.at[srcbuf.at[pl.ds(u * CK + j * 128, 128)]],
                add=True)

    def block(b, _):
        off = ebase + b * B
        pltpu.sync_copy(edge_hbm.at[0, pl.ds(off, B)], srcbuf)
        pltpu.sync_copy(edge_hbm.at[1, pl.ds(off, B)], dstbuf)
        g_cps = issue_gathers(0)
        s_cps = [None, None]
        for u in range(UPB):
            if u + 1 < UPB:
                if s_cps[(u + 1) % 2] is not None:
                    for cp in s_cps[(u + 1) % 2]:
                        cp.wait()
                ng = issue_gathers(u + 1)
            for cp in g_cps:
                cp.wait()
            s_cps[u % 2] = issue_scatters(u)
            do_hist(u)
            if u + 1 < UPB:
                g_cps = ng
        for slot in range(2):
            for cp in s_cps[slot]:
                cp.wait()
        return _
    lax.fori_loop(0, NBLK, block, None)

    def fill_tail(r, _):
        srcbuf[pl.ds(REM + r * 16, 16)] = jnp.zeros((16,), jnp.int32)
        dstbuf[pl.ds(REM + r * 16, 16)] = jnp.full((16,), N, jnp.int32)
        return _
    lax.fori_loop(0, (4 * CK - REM) // 16, fill_tail, None)
    roff = ebase + NBLK * B
    pltpu.sync_copy(edge_hbm.at[0, pl.ds(roff, REM)], srcbuf.at[pl.ds(0, REM)])
    pltpu.sync_copy(edge_hbm.at[1, pl.ds(roff, REM)], dstbuf.at[pl.ds(0, REM)])
    for u in range(4):
        for cp in issue_gathers(u):
            cp.wait()
        for cp in issue_scatters(u):
            cp.wait()
        do_hist(u)

    plsc.subcore_barrier()

    pltpu.sync_copy(histsp.at[pl.ds(s * ROWS_PER_TILE, ROWS_PER_TILE)], degbuf)
    pltpu.sync_copy(b1_hbm.at[c], b1buf)

    @pl.when(s == 0)
    def _():
        v = degbuf[pl.ds(0, 16)]
        lane = lax.iota(jnp.int32, 16)
        degbuf[pl.ds(0, 16)] = v - jnp.where(lane == 0, PAD_CNT, 0.0)

    b1lo = b1buf[pl.ds(0, 16)]
    b1hi = b1buf[pl.ds(16, 16)]
    nrows = jnp.minimum(ROWS_PER_TILE, N - s * ROWS_PER_TILE)

    def weigh_rows(carry_chunk):
        k, nch = carry_chunk
        rcount = jnp.clip(nrows - k * 512, 0, nch)

        def row(r, sacc):
            a0, a1 = sacc
            dj = plsc.load_gather(degbuf, [jnp.full((16,), k * 512 + r,
                                                    jnp.int32)])
            m0 = msg[r, pl.ds(0, 16)]
            m1 = msg[r, pl.ds(16, 16)]
            a0 = a0 + dj * jnp.maximum(m0 + b1lo, 0.0)
            a1 = a1 + dj * jnp.maximum(m1 + b1hi, 0.0)
            return (a0, a1)
        return rcount, row

    acc0 = jnp.zeros((16,), jnp.float32)
    acc1 = jnp.zeros((16,), jnp.float32)
    for k in range(7):
        nch = 512 if k < 6 else 64
        pltpu.sync_copy(acc.at[pl.ds(s * ROWS_PER_TILE + k * 512, nch)],
                        msg.at[pl.ds(0, nch)])
        rcount, row = weigh_rows((k, nch))
        acc0, acc1 = lax.fori_loop(0, rcount, row, (acc0, acc1))

    ones[pl.ds(0, 16)] = acc0
    ones[pl.ds(16, 16)] = acc1
    pltpu.sync_copy(ones.at[pl.ds(0, 32)], s_hbm.at[c, s])


def _sc_aggregate(y, edge_index, b1):
    mesh = plsc.VectorSubcoreMesh(core_axis_name="c", subcore_axis_name="s")
    fn = pl.kernel(
        _sc_body,
        out_type=jax.ShapeDtypeStruct((2, 16, H), jnp.float32),
        mesh=mesh,
        compiler_params=pltpu.CompilerParams(
            needs_layout_passes=False, use_tc_tiling_on_sc=False),
        scratch_types=[
            pltpu.VMEM_SHARED((NP, H), jnp.float32),
            pltpu.VMEM_SHARED((NP,), jnp.float32),
            pltpu.VMEM((B,), jnp.int32),
            pltpu.VMEM((B,), jnp.int32),
            pltpu.VMEM((K, H), jnp.float32),
            pltpu.VMEM((128,), jnp.float32),
            pltpu.VMEM((ROWS_PER_TILE,), jnp.float32),
            pltpu.VMEM((H,), jnp.float32),
            [pltpu.SemaphoreType.DMA] * 2,
            [pltpu.SemaphoreType.DMA] * 2,
        ],
    )
    return fn(y, edge_index, b1)


def _finish_body(s_ref, w2a_ref, w2b_ref, b2_ref, out_ref):
    s0 = jnp.sum(s_ref[0], axis=0, keepdims=True)
    s1 = jnp.sum(s_ref[1], axis=0, keepdims=True)
    out = (jnp.dot(s0, w2a_ref[...], preferred_element_type=jnp.float32)
           + jnp.dot(s1, w2b_ref[...], preferred_element_type=jnp.float32))
    out_ref[...] = out * (1.0 / N) + b2_ref[...]


def _finish(sp, w2a, w2b, b2):
    return pl.pallas_call(
        _finish_body,
        out_shape=jax.ShapeDtypeStruct((1, D), jnp.float32),
    )(sp, w2a, w2b, b2)


@jax.jit
def kernel(feats, edge_index, W1, b1, W2, b2):
    xp = feats.reshape(N // 4, 4 * D)
    z = jnp.zeros((4 * D, 128), jnp.float32)
    bd0 = z
    bd1 = z
    for k in range(4):
        bd0 = bd0.at[k * D:(k + 1) * D, k * H:(k + 1) * H].set(W1[:, :H])
        bd1 = bd1.at[k * D:(k + 1) * D, k * H:(k + 1) * H].set(W1[:, H:])
    y = _project(xp, bd0, bd1).reshape(2, NP, H)
    sp = _sc_aggregate(y, edge_index, b1.reshape(2, H))
    return _finish(sp, W2[:H, :], W2[H:, :], b2.reshape(1, D))
